# Initial kernel scaffold; baseline (speedup 1.0000x reference)
import jax
import jax.numpy as jnp
from jax.experimental import pallas as pl


def kernel(h, e, edge_index, Wq, bq, Wk, bk, Wv, bv, We, be):
    raise NotImplementedError("write your pallas kernel here")


# speedup vs baseline: 19.7716x; 19.7716x over previous
import functools

import jax
import jax.numpy as jnp
from jax import lax
from jax.experimental import pallas as pl
from jax.experimental.pallas import tpu as pltpu
from jax.experimental.pallas import tpu_sc as plsc

N_NODES = 10000
N_EDGES = 320000
F = 128
H = 8
D = 16

NC = 2
NS = 16
NW = NC * NS
CHUNK = 32
N_CHUNKS = N_EDGES // CHUNK
MAX_ITERS = (N_CHUNKS + NW - 1) // NW
ROWC = 40
N_WV_CHUNKS = N_NODES // ROWC
ZROWS = 1280
N_Z_CHUNKS = ZROWS // ROWC


def _proj_kernel(x_ref, w_ref, b_ref, o_ref):
    o_ref[...] = (
        jnp.dot(x_ref[...], w_ref[...], preferred_element_type=jnp.float32)
        + b_ref[...]
    )


def _project(x, w, b, block_rows):
    rows = x.shape[0]
    cols = w.shape[1]
    grid = rows // block_rows
    return pl.pallas_call(
        _proj_kernel,
        grid=(grid,),
        in_specs=[
            pl.BlockSpec((block_rows, x.shape[1]), lambda i: (i, 0)),
            pl.BlockSpec((w.shape[0], cols), lambda i: (0, 0)),
            pl.BlockSpec((1, cols), lambda i: (0, 0)),
        ],
        out_specs=pl.BlockSpec((block_rows, cols), lambda i: (i, 0)),
        out_shape=jax.ShapeDtypeStruct((rows, cols), jnp.float32),
    )(x, w, b)


def _combine_kernel(wv_ref, z_ref, o_ref):
    wv = wv_ref[0] + wv_ref[1]
    z = z_ref[0, :, :H] + z_ref[1, :, :H]
    zr = jnp.repeat(z + 1e-6, D, axis=1)
    o_ref[...] = wv / zr


def _combine(wvp, z16, block_rows):
    grid = N_NODES // block_rows
    return pl.pallas_call(
        _combine_kernel,
        grid=(grid,),
        in_specs=[
            pl.BlockSpec((2, block_rows, F), lambda i: (0, i, 0)),
            pl.BlockSpec((2, block_rows, 16), lambda i: (0, i, 0)),
        ],
        out_specs=pl.BlockSpec((block_rows, F), lambda i: (i, 0)),
        out_shape=jax.ShapeDtypeStruct((N_NODES, F), jnp.float32),
    )(wvp, z16)


_GATHER_DN = lax.GatherDimensionNumbers(
    offset_dims=(), collapsed_slice_dims=(0,), start_index_map=(0,)
)


def _lane_take(v, idx):
    return lax.gather(
        v, idx[:, None], _GATHER_DN, slice_sizes=(1,),
        mode=lax.GatherScatterMode.PROMISE_IN_BOUNDS,
    )


def _sc_edge_body(
    k_hbm, q_hbm, v_hbm, ef_hbm, src_hbm, dst_hbm,
    eout_hbm, wvp_hbm, zp_hbm,
    src_v, dst_v, zrow_v, slot_v, krows, qrows, vrows, efrows, sums, zbuf,
    wv_acc, z_acc, semk, semq, semv, seme,
):
    c = lax.axis_index("c")
    s = lax.axis_index("s")
    wid = s * NC + c

    zero16 = jnp.zeros((16,), jnp.float32)

    def _zero_zbuf(j, carry):
        zbuf[j // 8, pl.ds((j % 8) * 16, 16)] = zero16
        return carry

    lax.fori_loop(0, ROWC * 8, _zero_zbuf, 0)

    def _row_chunks(n_chunks, fn):
        def _body(j, cy):
            idx = s + NS * j

            @pl.when(idx < n_chunks)
            def _():
                fn(idx * ROWC)

            return cy

        lax.fori_loop(0, (n_chunks + NS - 1) // NS, _body, 0)

    _row_chunks(N_WV_CHUNKS,
                lambda r: pltpu.sync_copy(zbuf, wv_acc.at[pl.ds(r, ROWC)]))
    _row_chunks(N_Z_CHUNKS,
                lambda r: pltpu.sync_copy(zbuf, z_acc.at[pl.ds(r, ROWC)]))
    plsc.subcore_barrier()

    lane = lax.iota(jnp.int32, 16)

    def _chunk(i, carry):
        g = i * NW + wid

        @pl.when(g < N_CHUNKS)
        def _():
            base = g * CHUNK
            pltpu.sync_copy(src_hbm.at[pl.ds(base, CHUNK)], src_v)
            pltpu.sync_copy(dst_hbm.at[pl.ds(base, CHUNK)], dst_v)
            ck = pltpu.async_copy(k_hbm.at[src_v], krows, semk)
            cq = pltpu.async_copy(q_hbm.at[dst_v], qrows, semq)
            cv = pltpu.async_copy(v_hbm.at[src_v], vrows, semv)
            ce = pltpu.async_copy(ef_hbm.at[pl.ds(base, CHUNK)], efrows, seme)

            for t in range(CHUNK // 16):
                d16 = dst_v[pl.ds(t * 16, 16)]
                zrow_v[pl.ds(t * 16, 16)] = lax.shift_right_logical(d16, 3)
                slot_v[pl.ds(t * 16, 16)] = (d16 & 7) * 16

            ck.wait()
            cq.wait()
            cv.wait()
            ce.wait()

            last = jnp.full((16,), 15, jnp.int32)

            def _edge(ei, cy):
                acc = jnp.zeros((16,), jnp.float32)
                for hd in range(H):
                    sl = pl.ds(hd * D, D)
                    sc = (krows[ei, sl] * qrows[ei, sl] * 0.25) * efrows[ei, sl]
                    efrows[ei, sl] = sc
                    tot = _lane_take(plsc.cumsum(sc), last)
                    acc = jnp.where(lane == hd, tot, acc)
                srow = jnp.exp(jnp.clip(acc, -5.0, 5.0))
                for hd in range(H):
                    sl = pl.ds(hd * D, D)
                    sv = _lane_take(srow, jnp.full((16,), hd, jnp.int32))
                    vrows[ei, sl] = vrows[ei, sl] * sv
                slot = plsc.load_gather(
                    slot_v, [jnp.full((16,), ei, jnp.int32)]
                )
                for s8 in range(8):
                    sums[ei, pl.ds(s8 * 16, 16)] = jnp.where(
                        slot == s8 * 16, srow, zero16
                    )
                return cy

            lax.fori_loop(0, CHUNK, _edge, 0)

            pltpu.sync_copy(efrows, eout_hbm.at[pl.ds(base, CHUNK)])
            pltpu.sync_copy(vrows, wv_acc.at[dst_v], add=True)
            pltpu.sync_copy(sums, z_acc.at[zrow_v], add=True)

        return carry

    lax.fori_loop(0, MAX_ITERS, _chunk, 0)
    plsc.subcore_barrier()

    _row_chunks(
        N_WV_CHUNKS,
        lambda r: pltpu.sync_copy(
            wv_acc.at[pl.ds(r, ROWC)], wvp_hbm.at[c, pl.ds(r, ROWC)]
        ),
    )
    _row_chunks(
        N_Z_CHUNKS,
        lambda r: pltpu.sync_copy(
            z_acc.at[pl.ds(r, ROWC)], zp_hbm.at[c, pl.ds(r, ROWC)]
        ),
    )


def _sc_edge(k, q, v, ef, src, dst):
    mesh = plsc.VectorSubcoreMesh(
        core_axis_name="c", subcore_axis_name="s", num_cores=NC,
        num_subcores=NS,
    )
    fn = functools.partial(
        pl.kernel,
        out_type=[
            jax.ShapeDtypeStruct((N_EDGES, F), jnp.float32),
            jax.ShapeDtypeStruct((NC, N_NODES, F), jnp.float32),
            jax.ShapeDtypeStruct((NC, ZROWS, F), jnp.float32),
        ],
        mesh=mesh,
        scratch_types=[
            pltpu.VMEM((CHUNK,), jnp.int32),
            pltpu.VMEM((CHUNK,), jnp.int32),
            pltpu.VMEM((CHUNK,), jnp.int32),
            pltpu.VMEM((CHUNK,), jnp.int32),
            pltpu.VMEM((CHUNK, F), jnp.float32),
            pltpu.VMEM((CHUNK, F), jnp.float32),
            pltpu.VMEM((CHUNK, F), jnp.float32),
            pltpu.VMEM((CHUNK, F), jnp.float32),
            pltpu.VMEM((CHUNK, F), jnp.float32),
            pltpu.VMEM((ROWC, F), jnp.float32),
            pltpu.VMEM_SHARED((N_NODES, F), jnp.float32),
            pltpu.VMEM_SHARED((ZROWS, F), jnp.float32),
            pltpu.SemaphoreType.DMA,
            pltpu.SemaphoreType.DMA,
            pltpu.SemaphoreType.DMA,
            pltpu.SemaphoreType.DMA,
        ],
        compiler_params=pltpu.CompilerParams(needs_layout_passes=False),
    )(_sc_edge_body)
    return fn(k, q, v, ef, src, dst)


def kernel(h, e, edge_index, Wq, bq, Wk, bk, Wv, bv, We, be):
    w_qkv = jnp.concatenate([Wq, Wk, Wv], axis=1)
    b_qkv = jnp.concatenate([bq, bk, bv]).reshape(1, 384)
    qkv = _project(h, w_qkv, b_qkv, block_rows=1000)
    q_t = qkv[:, :F]
    k_t = qkv[:, F:2 * F]
    v_t = qkv[:, 2 * F:]

    ef = _project(e, We, be.reshape(1, F), block_rows=2000)

    src = edge_index[0]
    dst = edge_index[1]
    e_out, wvp, zp = _sc_edge(k_t, q_t, v_t, ef, src, dst)

    z16 = zp[:, : N_NODES // 8, :].reshape(NC, N_NODES, 16)
    h_out = _combine(wvp, z16, block_rows=1000)
    return h_out.reshape(N_NODES, H, D), e_out.reshape(N_EDGES, H, D)


# speedup vs baseline: 25.2403x; 1.2766x over previous
import functools

import jax
import jax.numpy as jnp
from jax import lax
from jax.experimental import pallas as pl
from jax.experimental.pallas import tpu as pltpu
from jax.experimental.pallas import tpu_sc as plsc

N_NODES = 10000
N_EDGES = 320000
F = 128
H = 8
D = 16

NC = 2
NS = 16
NW = NC * NS
CHUNK = 32
N_CHUNKS = N_EDGES // CHUNK
MAX_ITERS = (N_CHUNKS + NW - 1) // NW
N_PAIRS = (MAX_ITERS + 1) // 2
ROWC = 16
N_WV_CHUNKS = N_NODES // ROWC
ZROWS = 1280
N_Z_CHUNKS = ZROWS // ROWC


def _proj_kernel(x_ref, w_ref, b_ref, o_ref):
    o_ref[...] = (
        jnp.dot(x_ref[...], w_ref[...], preferred_element_type=jnp.float32)
        + b_ref[...]
    )


def _project(x, w, b, block_rows):
    rows = x.shape[0]
    cols = w.shape[1]
    grid = rows // block_rows
    return pl.pallas_call(
        _proj_kernel,
        grid=(grid,),
        in_specs=[
            pl.BlockSpec((block_rows, x.shape[1]), lambda i: (i, 0)),
            pl.BlockSpec((w.shape[0], cols), lambda i: (0, 0)),
            pl.BlockSpec((1, cols), lambda i: (0, 0)),
        ],
        out_specs=pl.BlockSpec((block_rows, cols), lambda i: (i, 0)),
        out_shape=jax.ShapeDtypeStruct((rows, cols), jnp.float32),
    )(x, w, b)


def _combine_kernel(wv_ref, z_ref, o_ref):
    wv = wv_ref[0] + wv_ref[1]
    z = z_ref[0, :, :H] + z_ref[1, :, :H]
    zr = jnp.repeat(z + 1e-6, D, axis=1)
    o_ref[...] = wv / zr


def _combine(wvp, z16, block_rows):
    grid = N_NODES // block_rows
    return pl.pallas_call(
        _combine_kernel,
        grid=(grid,),
        in_specs=[
            pl.BlockSpec((2, block_rows, F), lambda i: (0, i, 0)),
            pl.BlockSpec((2, block_rows, 16), lambda i: (0, i, 0)),
        ],
        out_specs=pl.BlockSpec((block_rows, F), lambda i: (i, 0)),
        out_shape=jax.ShapeDtypeStruct((N_NODES, F), jnp.float32),
    )(wvp, z16)


_GATHER_DN = lax.GatherDimensionNumbers(
    offset_dims=(), collapsed_slice_dims=(0,), start_index_map=(0,)
)


def _lane_take(v, idx):
    return lax.gather(
        v, idx[:, None], _GATHER_DN, slice_sizes=(1,),
        mode=lax.GatherScatterMode.PROMISE_IN_BOUNDS,
    )


def _sc_edge_body(
    k_hbm, q_hbm, v_hbm, ef_hbm, src_hbm, dst_hbm,
    eout_hbm, wvp_hbm, zp_hbm,
    srcv0, srcv1, dstv0, dstv1, zrow0, zrow1, slot0, slot1,
    kr0, kr1, qr0, qr1, vr0, vr1, efr0, efr1, sums,
    wv_acc, z_acc,
    semk0, semk1, semq0, semq1, semv0, semv1, seme0, seme1, semi0, semi1,
):
    srcv = [srcv0, srcv1]
    dstv = [dstv0, dstv1]
    zrow = [zrow0, zrow1]
    slotv = [slot0, slot1]
    kr = [kr0, kr1]
    qr = [qr0, qr1]
    vr = [vr0, vr1]
    efr = [efr0, efr1]
    semk = [semk0, semk1]
    semq = [semq0, semq1]
    semv = [semv0, semv1]
    seme = [seme0, seme1]
    semi = [semi0, semi1]

    c = lax.axis_index("c")
    s = lax.axis_index("s")
    wid = s * NC + c

    zero16 = jnp.zeros((16,), jnp.float32)
    lane = lax.iota(jnp.int32, 16)
    last = jnp.full((16,), 15, jnp.int32)

    def _zero_sums(j, carry):
        sums[j // 8, pl.ds((j % 8) * 16, 16)] = zero16
        return carry

    lax.fori_loop(0, CHUNK * 8, _zero_sums, 0)

    def _row_chunks(n_chunks, fn):
        def _body(j, cy):
            idx = s + NS * j

            @pl.when(idx < n_chunks)
            def _():
                fn(idx * ROWC)

            return cy

        lax.fori_loop(0, (n_chunks + NS - 1) // NS, _body, 0)

    _row_chunks(
        N_WV_CHUNKS,
        lambda r: pltpu.sync_copy(
            sums.at[pl.ds(0, ROWC)], wv_acc.at[pl.ds(r, ROWC)]
        ),
    )
    _row_chunks(
        N_Z_CHUNKS,
        lambda r: pltpu.sync_copy(
            sums.at[pl.ds(0, ROWC)], z_acc.at[pl.ds(r, ROWC)]
        ),
    )
    plsc.subcore_barrier()

    def _idx_issue(b, base):
        pltpu.async_copy(src_hbm.at[pl.ds(base, CHUNK)], srcv[b], semi[b])
        pltpu.async_copy(dst_hbm.at[pl.ds(base, CHUNK)], dstv[b], semi[b])

    def _idx_wait(b, base):
        pltpu.make_async_copy(
            src_hbm.at[pl.ds(base, CHUNK)], srcv[b], semi[b]
        ).wait()
        pltpu.make_async_copy(
            dst_hbm.at[pl.ds(base, CHUNK)], dstv[b], semi[b]
        ).wait()

    def _zrow_slot(b):
        for t in range(CHUNK // 16):
            sl = pl.ds(t * 16, 16)
            d16 = dstv[b][sl]
            zrow[b][sl] = lax.shift_right_logical(d16, 3)
            slotv[b][sl] = (d16 & 7) * 16

    def _gather_issue(b, base):
        pltpu.async_copy(k_hbm.at[srcv[b]], kr[b], semk[b])
        pltpu.async_copy(q_hbm.at[dstv[b]], qr[b], semq[b])
        pltpu.async_copy(v_hbm.at[srcv[b]], vr[b], semv[b])
        pltpu.async_copy(ef_hbm.at[pl.ds(base, CHUNK)], efr[b], seme[b])

    def _gather_wait(b, base):
        pltpu.make_async_copy(k_hbm.at[srcv[b]], kr[b], semk[b]).wait()
        pltpu.make_async_copy(q_hbm.at[dstv[b]], qr[b], semq[b]).wait()
        pltpu.make_async_copy(v_hbm.at[srcv[b]], vr[b], semv[b]).wait()
        pltpu.make_async_copy(
            ef_hbm.at[pl.ds(base, CHUNK)], efr[b], seme[b]
        ).wait()

    def _compute(b):
        krows, qrows, vrows, efrows = kr[b], qr[b], vr[b], efr[b]
        slot_r = slotv[b]

        def _edge(ei, cy):
            acc = jnp.zeros((16,), jnp.float32)
            for hd in range(H):
                sl = pl.ds(hd * D, D)
                sc = (krows[ei, sl] * qrows[ei, sl] * 0.25) * efrows[ei, sl]
                efrows[ei, sl] = sc
                tot = _lane_take(plsc.cumsum(sc), last)
                acc = jnp.where(lane == hd, tot, acc)
            srow = jnp.exp(jnp.clip(acc, -5.0, 5.0))
            for hd in range(H):
                sl = pl.ds(hd * D, D)
                sv = _lane_take(srow, jnp.full((16,), hd, jnp.int32))
                vrows[ei, sl] = vrows[ei, sl] * sv
            slot = plsc.load_gather(slot_r, [jnp.full((16,), ei, jnp.int32)])
            for s8 in range(8):
                sums[ei, pl.ds(s8 * 16, 16)] = jnp.where(
                    slot == s8 * 16, srow, zero16
                )
            return cy

        lax.fori_loop(0, CHUNK, _edge, 0)

    g0 = wid
    pltpu.sync_copy(src_hbm.at[pl.ds(g0 * CHUNK, CHUNK)], srcv[0])
    pltpu.sync_copy(dst_hbm.at[pl.ds(g0 * CHUNK, CHUNK)], dstv[0])
    _zrow_slot(0)
    _gather_issue(0, g0 * CHUNK)
    _idx_issue(1, (g0 + NW) * CHUNK)

    def _sub_iter(i, b):
        nb = 1 - b
        g = i * NW + wid
        gn = g + NW
        gi = g + 2 * NW

        @pl.when(gn < N_CHUNKS)
        def _():
            _idx_wait(nb, gn * CHUNK)
            _zrow_slot(nb)
            _gather_issue(nb, gn * CHUNK)

        @pl.when(g < N_CHUNKS)
        def _():
            base = g * CHUNK
            _gather_wait(b, base)
            _compute(b)
            pltpu.sync_copy(efr[b], eout_hbm.at[pl.ds(base, CHUNK)])
            pltpu.sync_copy(vr[b], wv_acc.at[dstv[b]], add=True)
            pltpu.sync_copy(sums, z_acc.at[zrow[b]], add=True)

        @pl.when(gi < N_CHUNKS)
        def _():
            _idx_issue(b, gi * CHUNK)

    def _pair(i2, carry):
        _sub_iter(i2 * 2, 0)
        _sub_iter(i2 * 2 + 1, 1)
        return carry

    lax.fori_loop(0, N_PAIRS, _pair, 0)
    plsc.subcore_barrier()

    _row_chunks(
        N_WV_CHUNKS,
        lambda r: pltpu.sync_copy(
            wv_acc.at[pl.ds(r, ROWC)], wvp_hbm.at[c, pl.ds(r, ROWC)]
        ),
    )
    _row_chunks(
        N_Z_CHUNKS,
        lambda r: pltpu.sync_copy(
            z_acc.at[pl.ds(r, ROWC)], zp_hbm.at[c, pl.ds(r, ROWC)]
        ),
    )


def _sc_edge(k, q, v, ef, src, dst):
    mesh = plsc.VectorSubcoreMesh(
        core_axis_name="c", subcore_axis_name="s", num_cores=NC,
        num_subcores=NS,
    )
    idx_t = pltpu.VMEM((CHUNK,), jnp.int32)
    row_t = pltpu.VMEM((CHUNK, F), jnp.float32)
    fn = functools.partial(
        pl.kernel,
        out_type=[
            jax.ShapeDtypeStruct((N_EDGES, F), jnp.float32),
            jax.ShapeDtypeStruct((NC, N_NODES, F), jnp.float32),
            jax.ShapeDtypeStruct((NC, ZROWS, F), jnp.float32),
        ],
        mesh=mesh,
        scratch_types=(
            [idx_t] * 8 + [row_t] * 9
            + [
                pltpu.VMEM_SHARED((N_NODES, F), jnp.float32),
                pltpu.VMEM_SHARED((ZROWS, F), jnp.float32),
            ]
            + [pltpu.SemaphoreType.DMA] * 10
        ),
        compiler_params=pltpu.CompilerParams(needs_layout_passes=False),
    )(_sc_edge_body)
    return fn(k, q, v, ef, src, dst)


def kernel(h, e, edge_index, Wq, bq, Wk, bk, Wv, bv, We, be):
    w_qkv = jnp.concatenate([Wq, Wk, Wv], axis=1)
    b_qkv = jnp.concatenate([bq, bk, bv]).reshape(1, 384)
    qkv = _project(h, w_qkv, b_qkv, block_rows=1000)
    q_t = qkv[:, :F]
    k_t = qkv[:, F:2 * F]
    v_t = qkv[:, 2 * F:]

    ef = _project(e, We, be.reshape(1, F), block_rows=2000)

    src = edge_index[0]
    dst = edge_index[1]
    e_out, wvp, zp = _sc_edge(k_t, q_t, v_t, ef, src, dst)

    z16 = zp[:, : N_NODES // 8, :].reshape(NC, N_NODES, 16)
    h_out = _combine(wvp, z16, block_rows=1000)
    return h_out.reshape(N_NODES, H, D), e_out.reshape(N_EDGES, H, D)


# speedup vs baseline: 28.4122x; 1.1257x over previous
import functools

import jax
import jax.numpy as jnp
from jax import lax
from jax.experimental import pallas as pl
from jax.experimental.pallas import tpu as pltpu
from jax.experimental.pallas import tpu_sc as plsc

N_NODES = 10000
N_EDGES = 320000
F = 128
H = 8
D = 16

NC = 2
NS = 16
NW = NC * NS
CHUNK = 32
N_CHUNKS = N_EDGES // CHUNK
MAX_ITERS = (N_CHUNKS + NW - 1) // NW
N_PAIRS = (MAX_ITERS + 1) // 2
ROWC = 16
N_WV_CHUNKS = N_NODES // ROWC
ZROWS = 1280
N_Z_CHUNKS = ZROWS // ROWC


def _proj_kernel(x_ref, w_ref, b_ref, o_ref):
    o_ref[...] = (
        jnp.dot(x_ref[...], w_ref[...], preferred_element_type=jnp.float32)
        + b_ref[...]
    )


def _project(x, w, b, block_rows):
    rows = x.shape[0]
    cols = w.shape[1]
    grid = rows // block_rows
    return pl.pallas_call(
        _proj_kernel,
        grid=(grid,),
        in_specs=[
            pl.BlockSpec((block_rows, x.shape[1]), lambda i: (i, 0)),
            pl.BlockSpec((w.shape[0], cols), lambda i: (0, 0)),
            pl.BlockSpec((1, cols), lambda i: (0, 0)),
        ],
        out_specs=pl.BlockSpec((block_rows, cols), lambda i: (i, 0)),
        out_shape=jax.ShapeDtypeStruct((rows, cols), jnp.float32),
    )(x, w, b)


def _combine_kernel(wv_ref, z_ref, o_ref):
    wv = wv_ref[0] + wv_ref[1]
    z = z_ref[0, :, :H] + z_ref[1, :, :H]
    zr = jnp.repeat(z + 1e-6, D, axis=1)
    o_ref[...] = wv / zr


def _combine(wvp, z16, block_rows):
    grid = N_NODES // block_rows
    return pl.pallas_call(
        _combine_kernel,
        grid=(grid,),
        in_specs=[
            pl.BlockSpec((2, block_rows, F), lambda i: (0, i, 0)),
            pl.BlockSpec((2, block_rows, 16), lambda i: (0, i, 0)),
        ],
        out_specs=pl.BlockSpec((block_rows, F), lambda i: (i, 0)),
        out_shape=jax.ShapeDtypeStruct((N_NODES, F), jnp.float32),
    )(wvp, z16)


_GATHER_DN = lax.GatherDimensionNumbers(
    offset_dims=(), collapsed_slice_dims=(0,), start_index_map=(0,)
)


def _lane_take(v, idx):
    return lax.gather(
        v, idx[:, None], _GATHER_DN, slice_sizes=(1,),
        mode=lax.GatherScatterMode.PROMISE_IN_BOUNDS,
    )


def _sc_edge_body(
    k_hbm, q_hbm, v_hbm, ef_hbm, src_hbm, dst_hbm,
    eout_hbm, wvp_hbm, zp_hbm,
    srcv0, srcv1, dstv0, dstv1, dsts0, dsts1, zrow0, zrow1, slot0, slot1,
    kr0, kr1, qr0, qr1, vr0, vr1, efr0, efr1, sums,
    wv_acc, z_acc,
    semk0, semk1, semq0, semq1, semv0, semv1, seme0, seme1, semi0, semi1,
    semo0, semo1, semw0, semw1, semz0, semz1,
):
    srcv = [srcv0, srcv1]
    dstv = [dstv0, dstv1]
    dsts = [dsts0, dsts1]
    zrow = [zrow0, zrow1]
    slotv = [slot0, slot1]
    kr = [kr0, kr1]
    qr = [qr0, qr1]
    vr = [vr0, vr1]
    efr = [efr0, efr1]
    semk = [semk0, semk1]
    semq = [semq0, semq1]
    semv = [semv0, semv1]
    seme = [seme0, seme1]
    semi = [semi0, semi1]
    semo = [semo0, semo1]
    semw = [semw0, semw1]
    semz = [semz0, semz1]

    c = lax.axis_index("c")
    s = lax.axis_index("s")
    wid = s * NC + c

    zero16 = jnp.zeros((16,), jnp.float32)
    lane = lax.iota(jnp.int32, 16)
    last = jnp.full((16,), 15, jnp.int32)

    def _zero_sums(j, carry):
        sums[j // 8, pl.ds((j % 8) * 16, 16)] = zero16
        return carry

    lax.fori_loop(0, CHUNK * 8, _zero_sums, 0)

    def _row_chunks(n_chunks, fn):
        def _body(j, cy):
            idx = s + NS * j

            @pl.when(idx < n_chunks)
            def _():
                fn(idx * ROWC)

            return cy

        lax.fori_loop(0, (n_chunks + NS - 1) // NS, _body, 0)

    _row_chunks(
        N_WV_CHUNKS,
        lambda r: pltpu.sync_copy(
            sums.at[pl.ds(0, ROWC)], wv_acc.at[pl.ds(r, ROWC)]
        ),
    )
    _row_chunks(
        N_Z_CHUNKS,
        lambda r: pltpu.sync_copy(
            sums.at[pl.ds(0, ROWC)], z_acc.at[pl.ds(r, ROWC)]
        ),
    )
    plsc.subcore_barrier()

    def _idx_issue(b, base):
        pltpu.async_copy(src_hbm.at[pl.ds(base, CHUNK)], srcv[b], semi[b])
        pltpu.async_copy(dst_hbm.at[pl.ds(base, CHUNK)], dstv[b], semi[b])

    def _idx_wait(b, base):
        pltpu.make_async_copy(
            src_hbm.at[pl.ds(base, CHUNK)], srcv[b], semi[b]
        ).wait()
        pltpu.make_async_copy(
            dst_hbm.at[pl.ds(base, CHUNK)], dstv[b], semi[b]
        ).wait()

    def _zrow_slot(b):
        for t in range(CHUNK // 16):
            sl = pl.ds(t * 16, 16)
            d16 = dstv[b][sl]
            dsts[b][sl] = d16
            zrow[b][sl] = lax.shift_right_logical(d16, 3)
            slotv[b][sl] = (d16 & 7) * 16

    def _gather_issue(b, base):
        pltpu.async_copy(k_hbm.at[srcv[b]], kr[b], semk[b])
        pltpu.async_copy(q_hbm.at[dstv[b]], qr[b], semq[b])
        pltpu.async_copy(v_hbm.at[srcv[b]], vr[b], semv[b])
        pltpu.async_copy(ef_hbm.at[pl.ds(base, CHUNK)], efr[b], seme[b])

    def _gather_wait(b, base):
        pltpu.make_async_copy(k_hbm.at[srcv[b]], kr[b], semk[b]).wait()
        pltpu.make_async_copy(q_hbm.at[dstv[b]], qr[b], semq[b]).wait()
        pltpu.make_async_copy(v_hbm.at[srcv[b]], vr[b], semv[b]).wait()
        pltpu.make_async_copy(
            ef_hbm.at[pl.ds(base, CHUNK)], efr[b], seme[b]
        ).wait()

    def _compute(b):
        krows, qrows, vrows, efrows = kr[b], qr[b], vr[b], efr[b]
        slot_r = slotv[b]

        def _edge(ei, cy):
            acc = jnp.zeros((16,), jnp.float32)
            for hd in range(H):
                sl = pl.ds(hd * D, D)
                sc = (krows[ei, sl] * qrows[ei, sl] * 0.25) * efrows[ei, sl]
                efrows[ei, sl] = sc
                tot = _lane_take(plsc.cumsum(sc), last)
                acc = jnp.where(lane == hd, tot, acc)
            srow = jnp.exp(jnp.clip(acc, -5.0, 5.0))
            for hd in range(H):
                sl = pl.ds(hd * D, D)
                sv = _lane_take(srow, jnp.full((16,), hd, jnp.int32))
                vrows[ei, sl] = vrows[ei, sl] * sv
            slot = plsc.load_gather(slot_r, [jnp.full((16,), ei, jnp.int32)])
            for s8 in range(8):
                sums[ei, pl.ds(s8 * 16, 16)] = jnp.where(
                    slot == s8 * 16, srow, zero16
                )
            return cy

        lax.fori_loop(0, CHUNK, _edge, 0)

    g0 = wid
    pltpu.sync_copy(src_hbm.at[pl.ds(g0 * CHUNK, CHUNK)], srcv[0])
    pltpu.sync_copy(dst_hbm.at[pl.ds(g0 * CHUNK, CHUNK)], dstv[0])
    _zrow_slot(0)
    _gather_issue(0, g0 * CHUNK)
    _idx_issue(1, (g0 + NW) * CHUNK)

    def _wait_out(b):
        pltpu.make_async_copy(
            efr[b], eout_hbm.at[pl.ds(0, CHUNK)], semo[b]
        ).wait()
        pltpu.make_async_copy(vr[b], wv_acc.at[dsts[b]], semw[b]).wait()

    def _wait_z(b):
        pltpu.make_async_copy(sums, z_acc.at[zrow[b]], semz[b]).wait()

    def _sub_iter(i, b):
        nb = 1 - b
        g = i * NW + wid
        gn = g + NW
        gi = g + 2 * NW

        @pl.when(gn < N_CHUNKS)
        def _():
            _idx_wait(nb, gn * CHUNK)

            @pl.when(i > 0)
            def _():
                _wait_out(nb)
                _wait_z(nb)

            _zrow_slot(nb)
            _gather_issue(nb, gn * CHUNK)

        @pl.when(g < N_CHUNKS)
        def _():
            base = g * CHUNK

            @pl.when(jnp.logical_and(gn >= N_CHUNKS, i > 0))
            def _():
                _wait_z(nb)

            _gather_wait(b, base)
            _compute(b)
            pltpu.async_copy(efr[b], eout_hbm.at[pl.ds(base, CHUNK)], semo[b])
            pltpu.async_copy(vr[b], wv_acc.at[dsts[b]], semw[b], add=True)
            pltpu.async_copy(sums, z_acc.at[zrow[b]], semz[b], add=True)

        @pl.when(gi < N_CHUNKS)
        def _():
            _idx_issue(b, gi * CHUNK)

    def _pair(i2, carry):
        _sub_iter(i2 * 2, 0)
        _sub_iter(i2 * 2 + 1, 1)
        return carry

    lax.fori_loop(0, N_PAIRS, _pair, 0)
    _wait_out(0)
    _wait_out(1)
    n_my_chunks = MAX_ITERS - 1 + jnp.int32(wid < N_CHUNKS - (MAX_ITERS - 1) * NW)
    last_b = (n_my_chunks - 1) & 1

    @pl.when(last_b == 0)
    def _():
        _wait_z(0)

    @pl.when(last_b == 1)
    def _():
        _wait_z(1)

    plsc.subcore_barrier()

    _row_chunks(
        N_WV_CHUNKS,
        lambda r: pltpu.sync_copy(
            wv_acc.at[pl.ds(r, ROWC)], wvp_hbm.at[c, pl.ds(r, ROWC)]
        ),
    )
    _row_chunks(
        N_Z_CHUNKS,
        lambda r: pltpu.sync_copy(
            z_acc.at[pl.ds(r, ROWC)], zp_hbm.at[c, pl.ds(r, ROWC)]
        ),
    )


def _sc_edge(k, q, v, ef, src, dst):
    mesh = plsc.VectorSubcoreMesh(
        core_axis_name="c", subcore_axis_name="s", num_cores=NC,
        num_subcores=NS,
    )
    idx_t = pltpu.VMEM((CHUNK,), jnp.int32)
    row_t = pltpu.VMEM((CHUNK, F), jnp.float32)
    fn = functools.partial(
        pl.kernel,
        out_type=[
            jax.ShapeDtypeStruct((N_EDGES, F), jnp.float32),
            jax.ShapeDtypeStruct((NC, N_NODES, F), jnp.float32),
            jax.ShapeDtypeStruct((NC, ZROWS, F), jnp.float32),
        ],
        mesh=mesh,
        scratch_types=(
            [idx_t] * 10 + [row_t] * 9
            + [
                pltpu.VMEM_SHARED((N_NODES, F), jnp.float32),
                pltpu.VMEM_SHARED((ZROWS, F), jnp.float32),
            ]
            + [pltpu.SemaphoreType.DMA] * 16
        ),
        compiler_params=pltpu.CompilerParams(needs_layout_passes=False),
    )(_sc_edge_body)
    return fn(k, q, v, ef, src, dst)


def kernel(h, e, edge_index, Wq, bq, Wk, bk, Wv, bv, We, be):
    w_qkv = jnp.concatenate([Wq, Wk, Wv], axis=1)
    b_qkv = jnp.concatenate([bq, bk, bv]).reshape(1, 384)
    qkv = _project(h, w_qkv, b_qkv, block_rows=1000)
    q_t = qkv[:, :F]
    k_t = qkv[:, F:2 * F]
    v_t = qkv[:, 2 * F:]

    ef = _project(e, We, be.reshape(1, F), block_rows=2000)

    src = edge_index[0]
    dst = edge_index[1]
    e_out, wvp, zp = _sc_edge(k_t, q_t, v_t, ef, src, dst)

    z16 = zp[:, : N_NODES // 8, :].reshape(NC, N_NODES, 16)
    h_out = _combine(wvp, z16, block_rows=1000)
    return h_out.reshape(N_NODES, H, D), e_out.reshape(N_EDGES, H, D)


# speedup vs baseline: 28.6086x; 1.0069x over previous
import functools

import jax
import jax.numpy as jnp
from jax import lax
from jax.experimental import pallas as pl
from jax.experimental.pallas import tpu as pltpu
from jax.experimental.pallas import tpu_sc as plsc

N_NODES = 10000
N_EDGES = 320000
F = 128
H = 8
D = 16

NC = 2
NS = 16
NW = NC * NS
CHUNK = 32
N_CHUNKS = N_EDGES // CHUNK
MAX_ITERS = (N_CHUNKS + NW - 1) // NW
N_PAIRS = (MAX_ITERS + 1) // 2
ROWC = 16
N_WV_CHUNKS = N_NODES // ROWC
ZROWS = 1280
N_Z_CHUNKS = ZROWS // ROWC


def _proj_kernel(x_ref, w_ref, b_ref, o_ref):
    o_ref[...] = (
        jnp.dot(x_ref[...], w_ref[...], preferred_element_type=jnp.float32)
        + b_ref[...]
    )


def _project(x, w, b, block_rows):
    rows = x.shape[0]
    cols = w.shape[1]
    grid = rows // block_rows
    return pl.pallas_call(
        _proj_kernel,
        grid=(grid,),
        in_specs=[
            pl.BlockSpec((block_rows, x.shape[1]), lambda i: (i, 0)),
            pl.BlockSpec((w.shape[0], cols), lambda i: (0, 0)),
            pl.BlockSpec((1, cols), lambda i: (0, 0)),
        ],
        out_specs=pl.BlockSpec((block_rows, cols), lambda i: (i, 0)),
        out_shape=jax.ShapeDtypeStruct((rows, cols), jnp.float32),
    )(x, w, b)


def _proj3_kernel(x_ref, w_ref, b_ref, q_ref, k_ref, v_ref):
    r = (
        jnp.dot(x_ref[...], w_ref[...], preferred_element_type=jnp.float32)
        + b_ref[...]
    )
    q_ref[...] = r[:, :F]
    k_ref[...] = r[:, F:2 * F]
    v_ref[...] = r[:, 2 * F:]


def _project3(x, w, b, block_rows):
    rows = x.shape[0]
    grid = rows // block_rows
    out = jax.ShapeDtypeStruct((rows, F), jnp.float32)
    return pl.pallas_call(
        _proj3_kernel,
        grid=(grid,),
        in_specs=[
            pl.BlockSpec((block_rows, x.shape[1]), lambda i: (i, 0)),
            pl.BlockSpec((w.shape[0], 3 * F), lambda i: (0, 0)),
            pl.BlockSpec((1, 3 * F), lambda i: (0, 0)),
        ],
        out_specs=[
            pl.BlockSpec((block_rows, F), lambda i: (i, 0)),
            pl.BlockSpec((block_rows, F), lambda i: (i, 0)),
            pl.BlockSpec((block_rows, F), lambda i: (i, 0)),
        ],
        out_shape=[out, out, out],
    )(x, w, b)


def _combine_kernel(wv_ref, z_ref, o_ref):
    wv = wv_ref[0] + wv_ref[1]
    z = z_ref[0, :, :H] + z_ref[1, :, :H]
    zr = jnp.repeat(z + 1e-6, D, axis=1)
    o_ref[...] = wv / zr


def _combine(wvp, z16, block_rows):
    grid = N_NODES // block_rows
    return pl.pallas_call(
        _combine_kernel,
        grid=(grid,),
        in_specs=[
            pl.BlockSpec((2, block_rows, F), lambda i: (0, i, 0)),
            pl.BlockSpec((2, block_rows, 16), lambda i: (0, i, 0)),
        ],
        out_specs=pl.BlockSpec((block_rows, F), lambda i: (i, 0)),
        out_shape=jax.ShapeDtypeStruct((N_NODES, F), jnp.float32),
    )(wvp, z16)


_GATHER_DN = lax.GatherDimensionNumbers(
    offset_dims=(), collapsed_slice_dims=(0,), start_index_map=(0,)
)


def _lane_take(v, idx):
    return lax.gather(
        v, idx[:, None], _GATHER_DN, slice_sizes=(1,),
        mode=lax.GatherScatterMode.PROMISE_IN_BOUNDS,
    )


def _sc_edge_body(
    k_hbm, q_hbm, v_hbm, ef_hbm, src_hbm, dst_hbm,
    eout_hbm, wvp_hbm, zp_hbm,
    srcv0, srcv1, dstv0, dstv1, dsts0, dsts1, zrow0, zrow1, slot0, slot1,
    kr0, kr1, qr0, qr1, vr0, vr1, efr0, efr1, sums,
    wv_acc, z_acc,
    semk0, semk1, semq0, semq1, semv0, semv1, seme0, seme1, semi0, semi1,
    semo0, semo1, semw0, semw1, semz0, semz1,
):
    srcv = [srcv0, srcv1]
    dstv = [dstv0, dstv1]
    dsts = [dsts0, dsts1]
    zrow = [zrow0, zrow1]
    slotv = [slot0, slot1]
    kr = [kr0, kr1]
    qr = [qr0, qr1]
    vr = [vr0, vr1]
    efr = [efr0, efr1]
    semk = [semk0, semk1]
    semq = [semq0, semq1]
    semv = [semv0, semv1]
    seme = [seme0, seme1]
    semi = [semi0, semi1]
    semo = [semo0, semo1]
    semw = [semw0, semw1]
    semz = [semz0, semz1]

    c = lax.axis_index("c")
    s = lax.axis_index("s")
    wid = s * NC + c

    zero16 = jnp.zeros((16,), jnp.float32)
    lane = lax.iota(jnp.int32, 16)
    last = jnp.full((16,), 15, jnp.int32)

    def _zero_sums(j, carry):
        sums[j // 8, pl.ds((j % 8) * 16, 16)] = zero16
        return carry

    lax.fori_loop(0, CHUNK * 8, _zero_sums, 0)

    def _row_chunks(n_chunks, fn):
        def _body(j, cy):
            idx = s + NS * j

            @pl.when(idx < n_chunks)
            def _():
                fn(idx * ROWC)

            return cy

        lax.fori_loop(0, (n_chunks + NS - 1) // NS, _body, 0)

    _row_chunks(
        N_WV_CHUNKS,
        lambda r: pltpu.sync_copy(
            sums.at[pl.ds(0, ROWC)], wv_acc.at[pl.ds(r, ROWC)]
        ),
    )
    _row_chunks(
        N_Z_CHUNKS,
        lambda r: pltpu.sync_copy(
            sums.at[pl.ds(0, ROWC)], z_acc.at[pl.ds(r, ROWC)]
        ),
    )
    plsc.subcore_barrier()

    def _idx_issue(b, base):
        pltpu.async_copy(src_hbm.at[pl.ds(base, CHUNK)], srcv[b], semi[b])
        pltpu.async_copy(dst_hbm.at[pl.ds(base, CHUNK)], dstv[b], semi[b])

    def _idx_wait(b, base):
        pltpu.make_async_copy(
            src_hbm.at[pl.ds(base, CHUNK)], srcv[b], semi[b]
        ).wait()
        pltpu.make_async_copy(
            dst_hbm.at[pl.ds(base, CHUNK)], dstv[b], semi[b]
        ).wait()

    def _zrow_slot(b):
        for t in range(CHUNK // 16):
            sl = pl.ds(t * 16, 16)
            d16 = dstv[b][sl]
            dsts[b][sl] = d16
            zrow[b][sl] = lax.shift_right_logical(d16, 3)
            slotv[b][sl] = (d16 & 7) * 16

    def _gather_issue(b, base):
        pltpu.async_copy(k_hbm.at[srcv[b]], kr[b], semk[b])
        pltpu.async_copy(q_hbm.at[dstv[b]], qr[b], semq[b])
        pltpu.async_copy(v_hbm.at[srcv[b]], vr[b], semv[b])
        pltpu.async_copy(ef_hbm.at[pl.ds(base, CHUNK)], efr[b], seme[b])

    def _gather_wait(b, base):
        pltpu.make_async_copy(k_hbm.at[srcv[b]], kr[b], semk[b]).wait()
        pltpu.make_async_copy(q_hbm.at[dstv[b]], qr[b], semq[b]).wait()
        pltpu.make_async_copy(v_hbm.at[srcv[b]], vr[b], semv[b]).wait()
        pltpu.make_async_copy(
            ef_hbm.at[pl.ds(base, CHUNK)], efr[b], seme[b]
        ).wait()

    def _compute(b):
        krows, qrows, vrows, efrows = kr[b], qr[b], vr[b], efr[b]
        slot_r = slotv[b]

        def _edge(ei, cy):
            acc = jnp.zeros((16,), jnp.float32)
            for hd in range(H):
                sl = pl.ds(hd * D, D)
                sc = (krows[ei, sl] * qrows[ei, sl] * 0.25) * efrows[ei, sl]
                efrows[ei, sl] = sc
                tot = _lane_take(plsc.cumsum(sc), last)
                acc = jnp.where(lane == hd, tot, acc)
            srow = jnp.exp(jnp.clip(acc, -5.0, 5.0))
            for hd in range(H):
                sl = pl.ds(hd * D, D)
                sv = _lane_take(srow, jnp.full((16,), hd, jnp.int32))
                vrows[ei, sl] = vrows[ei, sl] * sv
            slot = plsc.load_gather(slot_r, [jnp.full((16,), ei, jnp.int32)])
            for s8 in range(8):
                sums[ei, pl.ds(s8 * 16, 16)] = jnp.where(
                    slot == s8 * 16, srow, zero16
                )
            return cy

        lax.fori_loop(0, CHUNK, _edge, 0)

    g0 = wid
    pltpu.sync_copy(src_hbm.at[pl.ds(g0 * CHUNK, CHUNK)], srcv[0])
    pltpu.sync_copy(dst_hbm.at[pl.ds(g0 * CHUNK, CHUNK)], dstv[0])
    _zrow_slot(0)
    _gather_issue(0, g0 * CHUNK)
    _idx_issue(1, (g0 + NW) * CHUNK)

    def _wait_out(b):
        pltpu.make_async_copy(
            efr[b], eout_hbm.at[pl.ds(0, CHUNK)], semo[b]
        ).wait()
        pltpu.make_async_copy(vr[b], wv_acc.at[dsts[b]], semw[b]).wait()

    def _wait_z(b):
        pltpu.make_async_copy(sums, z_acc.at[zrow[b]], semz[b]).wait()

    def _sub_iter(i, b):
        nb = 1 - b
        g = i * NW + wid
        gn = g + NW
        gi = g + 2 * NW

        @pl.when(gn < N_CHUNKS)
        def _():
            _idx_wait(nb, gn * CHUNK)

            @pl.when(i > 0)
            def _():
                _wait_out(nb)
                _wait_z(nb)

            _zrow_slot(nb)
            _gather_issue(nb, gn * CHUNK)

        @pl.when(g < N_CHUNKS)
        def _():
            base = g * CHUNK

            @pl.when(jnp.logical_and(gn >= N_CHUNKS, i > 0))
            def _():
                _wait_z(nb)

            _gather_wait(b, base)
            _compute(b)
            pltpu.async_copy(efr[b], eout_hbm.at[pl.ds(base, CHUNK)], semo[b])
            pltpu.async_copy(vr[b], wv_acc.at[dsts[b]], semw[b], add=True)
            pltpu.async_copy(sums, z_acc.at[zrow[b]], semz[b], add=True)

        @pl.when(gi < N_CHUNKS)
        def _():
            _idx_issue(b, gi * CHUNK)

    def _pair(i2, carry):
        _sub_iter(i2 * 2, 0)
        _sub_iter(i2 * 2 + 1, 1)
        return carry

    lax.fori_loop(0, N_PAIRS, _pair, 0)
    _wait_out(0)
    _wait_out(1)
    n_my_chunks = MAX_ITERS - 1 + jnp.int32(wid < N_CHUNKS - (MAX_ITERS - 1) * NW)
    last_b = (n_my_chunks - 1) & 1

    @pl.when(last_b == 0)
    def _():
        _wait_z(0)

    @pl.when(last_b == 1)
    def _():
        _wait_z(1)

    plsc.subcore_barrier()

    _row_chunks(
        N_WV_CHUNKS,
        lambda r: pltpu.sync_copy(
            wv_acc.at[pl.ds(r, ROWC)], wvp_hbm.at[c, pl.ds(r, ROWC)]
        ),
    )
    _row_chunks(
        N_Z_CHUNKS,
        lambda r: pltpu.sync_copy(
            z_acc.at[pl.ds(r, ROWC)], zp_hbm.at[c, pl.ds(r, ROWC)]
        ),
    )


def _sc_edge(k, q, v, ef, src, dst):
    mesh = plsc.VectorSubcoreMesh(
        core_axis_name="c", subcore_axis_name="s", num_cores=NC,
        num_subcores=NS,
    )
    idx_t = pltpu.VMEM((CHUNK,), jnp.int32)
    row_t = pltpu.VMEM((CHUNK, F), jnp.float32)
    fn = functools.partial(
        pl.kernel,
        out_type=[
            jax.ShapeDtypeStruct((N_EDGES, F), jnp.float32),
            jax.ShapeDtypeStruct((NC, N_NODES, F), jnp.float32),
            jax.ShapeDtypeStruct((NC, ZROWS, F), jnp.float32),
        ],
        mesh=mesh,
        scratch_types=(
            [idx_t] * 10 + [row_t] * 9
            + [
                pltpu.VMEM_SHARED((N_NODES, F), jnp.float32),
                pltpu.VMEM_SHARED((ZROWS, F), jnp.float32),
            ]
            + [pltpu.SemaphoreType.DMA] * 16
        ),
        compiler_params=pltpu.CompilerParams(needs_layout_passes=False),
    )(_sc_edge_body)
    return fn(k, q, v, ef, src, dst)


def kernel(h, e, edge_index, Wq, bq, Wk, bk, Wv, bv, We, be):
    w_qkv = jnp.concatenate([Wq, Wk, Wv], axis=1)
    b_qkv = jnp.concatenate([bq, bk, bv]).reshape(1, 384)
    q_t, k_t, v_t = _project3(h, w_qkv, b_qkv, block_rows=1000)

    ef = _project(e, We, be.reshape(1, F), block_rows=2000)

    src = edge_index[0]
    dst = edge_index[1]
    e_out, wvp, zp = _sc_edge(k_t, q_t, v_t, ef, src, dst)

    z16 = zp[:, : N_NODES // 8, :].reshape(NC, N_NODES, 16)
    h_out = _combine(wvp, z16, block_rows=1000)
    return h_out.reshape(N_NODES, H, D), e_out.reshape(N_EDGES, H, D)
